# baseline (speedup 1.0000x reference)
import jax
import jax.numpy as jnp
from jax import lax
from jax.experimental import pallas as pl
from jax.experimental.pallas import tpu as pltpu
from jax.experimental.pallas import tpu_sc as plsc

N = 10000
K = 32
D = 128
H = 4
U = 32
HU = H * U

NW = 32
CH = 2
N_PAD = 10240
NODES_W = N_PAD // NW
CHUNKS_W = NODES_W // CH
NBUF = 2
GRP = 4
GUARD = 8
WIN = 5120
NEG = -1000000000.0


def _splat(val, dtype=jnp.float32):
    return jnp.full((16,), val, dtype=dtype)


_GDN = lax.GatherDimensionNumbers(
    offset_dims=(), collapsed_slice_dims=(0,), start_index_map=(0,))


def _gather16(vec, idx16):
    return lax.gather(vec, idx16[:, None], _GDN, (1,),
                      mode=lax.GatherScatterMode.PROMISE_IN_BOUNDS)


def _last_lane(vec):
    return _gather16(vec, _splat(15, jnp.int32))


def _mm_body(x_ref, w_ref, asx_ref, anx_ref, h_ref, ss_ref, sn_ref):
    h = jnp.dot(x_ref[...], w_ref[...], preferred_element_type=jnp.float32)
    h_ref[...] = h
    ss_ref[...] = jnp.dot(h, asx_ref[...], preferred_element_type=jnp.float32)
    sn_ref[...] = jnp.dot(h, anx_ref[...], preferred_element_type=jnp.float32)


def _project(x_pad, W, a_self_m, a_nbr_m):
    blk = 1024
    grid = N_PAD // blk
    return pl.pallas_call(
        _mm_body,
        grid=(grid,),
        in_specs=[
            pl.BlockSpec((blk, D), lambda i: (i, 0)),
            pl.BlockSpec((D, HU), lambda i: (0, 0)),
            pl.BlockSpec((HU, H), lambda i: (0, 0)),
            pl.BlockSpec((HU, H), lambda i: (0, 0)),
        ],
        out_specs=[
            pl.BlockSpec((blk, HU), lambda i: (i, 0)),
            pl.BlockSpec((blk, H), lambda i: (i, 0)),
            pl.BlockSpec((blk, H), lambda i: (i, 0)),
        ],
        out_shape=[
            jax.ShapeDtypeStruct((N_PAD, HU), jnp.float32),
            jax.ShapeDtypeStruct((N_PAD, H), jnp.float32),
            jax.ShapeDtypeStruct((N_PAD, H), jnp.float32),
        ],
    )(x_pad, W, a_self_m, a_nbr_m)


def _attn_body(hz_hbm, ss_hbm, sng_hbm, nbr_hbm, out_hbm,
               sng_v, sself_v, adj_v, isp_refs, ihbm_refs,
               rsp_refs, rhbm_refs, ctx_v, hz_sp, sems_sp, sems_hbm):
    cid = lax.axis_index("c")
    sid = lax.axis_index("s")
    wid = cid * 16 + sid
    node0 = wid * NODES_W
    start_c = cid * WIN

    rows_per_sub = WIN // 16
    pltpu.sync_copy(
        hz_hbm.at[pl.ds(start_c + 16 + sid * rows_per_sub, rows_per_sub)],
        hz_sp.at[pl.ds(8 + sid * rows_per_sub, rows_per_sub)])

    @pl.when(sid == 0)
    def _():
        pltpu.sync_copy(hz_hbm.at[pl.ds(0, 8)], hz_sp.at[pl.ds(0, 8)])

    pltpu.sync_copy(sng_hbm, sng_v)
    pltpu.sync_copy(ss_hbm.at[pl.ds(node0 * H, NODES_W * H)], sself_v)
    pltpu.sync_copy(nbr_hbm.at[pl.ds(node0 * K, NODES_W * K)], adj_v)
    plsc.subcore_barrier()

    vstart = _splat(start_c, jnp.int32)
    vend = _splat(start_c + WIN, jnp.int32)
    vzero = _splat(0, jnp.int32)
    vsp_off = _splat(7 - start_c, jnp.int32)
    vhbm_off = _splat(15, jnp.int32)
    vinv = _splat(8, jnp.int32)

    def issue(g, slot):
        for q in range(CH * 2):
            v = adj_v[pl.ds(g * CH * K + q * 16, 16)]
            in_win = (v > vstart) & (v <= vend)
            isp = jnp.where(in_win, v + vsp_off, vzero)
            ihbm = jnp.where(in_win, vzero,
                             jnp.where(v > vzero, v + vhbm_off, vinv))
            isp_refs[slot][pl.ds(q * 16, 16)] = isp
            ihbm_refs[slot][pl.ds(q * 16, 16)] = ihbm
        pltpu.make_async_copy(
            hz_sp.at[isp_refs[slot]], rsp_refs[slot], sems_sp[slot]).start()
        pltpu.make_async_copy(
            hz_hbm.at[ihbm_refs[slot]], rhbm_refs[slot], sems_hbm[slot]).start()

    def wait(slot):
        pltpu.make_async_copy(
            hz_sp.at[isp_refs[slot]], rsp_refs[slot], sems_sp[slot]).wait()
        pltpu.make_async_copy(
            hz_hbm.at[ihbm_refs[slot]], rhbm_refs[slot], sems_hbm[slot]).wait()

    def compute(g, slot, jrow0):
        rsp = rsp_refs[slot]
        rhbm = rhbm_refs[slot]
        for j in range(CH):
            nl = g * CH + j
            nbrs = [adj_v[pl.ds(nl * K + kc * 16, 16)] for kc in range(2)]
            evecs = []
            for h in range(H):
                sself = plsc.load_gather(
                    sself_v, [_splat(nl * H + h, jnp.int32)])
                scs = []
                for kc in range(2):
                    snbr = plsc.load_gather(
                        sng_v, [nbrs[kc] * H + (H + h)])
                    sc = sself + snbr
                    scs.append(jnp.where(sc > 0, sc, 0.2 * sc))
                mx = _last_lane(plsc.cummax(jnp.maximum(scs[0], scs[1])))
                e0 = jnp.exp(scs[0] - mx)
                e1 = jnp.exp(scs[1] - mx)
                rden = 1.0 / _last_lane(plsc.cumsum(e0 + e1))
                evecs.append((e0 * rden, e1 * rden))

            def kbody(kc):
                def body(it, accs):
                    base = it * 4
                    lane0 = _splat(base - kc * 16, jnp.int32)
                    out = list(accs)
                    ebs = [[_gather16(evecs[h][kc], lane0 + jj)
                            for jj in range(4)] for h in range(H)]
                    for jj in range(4):
                        row = j * K + base + jj
                        for h in range(H):
                            for uc in range(2):
                                c = h * 2 + uc
                                rv = (rsp[row, pl.ds(c * 16, 16)] +
                                      rhbm[row, pl.ds(c * 16, 16)])
                                out[c] = out[c] + ebs[h][jj] * rv
                    return tuple(out)
                return body

            accs = tuple(jnp.zeros((16,), jnp.float32) for _ in range(8))
            accs = lax.fori_loop(0, 4, kbody(0), accs)
            accs = lax.fori_loop(4, 8, kbody(1), accs)
            for c in range(8):
                ctx_v[jrow0 + j, pl.ds(c * 16, 16)] = accs[c]

    issue(0, 0)

    def outer(t, carry):
        g0 = t * GRP
        for b in range(GRP):
            g = g0 + b
            issue(jnp.minimum(g + 1, CHUNKS_W - 1), (b + 1) % NBUF)
            wait(b % NBUF)
            compute(g, b % NBUF, b * CH)
        pltpu.sync_copy(ctx_v, out_hbm.at[pl.ds(node0 + g0 * CH, GRP * CH)])
        return carry

    lax.fori_loop(0, CHUNKS_W // GRP, outer, 0)
    wait(0)


def _attention(hz, s_self, s_nbr_g, nbr_pad):
    mesh = plsc.VectorSubcoreMesh(core_axis_name="c", subcore_axis_name="s")
    kfn = pl.kernel(
        _attn_body,
        out_type=jax.ShapeDtypeStruct((N_PAD, HU), jnp.float32),
        mesh=mesh,
        scratch_types=[
            pltpu.VMEM((GUARD + N_PAD * H,), jnp.float32),
            pltpu.VMEM((NODES_W * H,), jnp.float32),
            pltpu.VMEM((NODES_W * K,), jnp.int32),
            [pltpu.VMEM((CH * K,), jnp.int32)] * NBUF,
            [pltpu.VMEM((CH * K,), jnp.int32)] * NBUF,
            [pltpu.VMEM((CH * K, HU), jnp.float32)] * NBUF,
            [pltpu.VMEM((CH * K, HU), jnp.float32)] * NBUF,
            pltpu.VMEM((GRP * CH, HU), jnp.float32),
            pltpu.VMEM_SHARED((WIN + 8, HU), jnp.float32),
            [pltpu.SemaphoreType.DMA] * NBUF,
            [pltpu.SemaphoreType.DMA] * NBUF,
        ],
        compiler_params=pltpu.CompilerParams(needs_layout_passes=False),
    )
    return kfn(hz, s_self.reshape(-1), s_nbr_g, nbr_pad.reshape(-1))


def _ln_body(x_ref, g_ref, b_ref, o_ref):
    y = jnp.maximum(x_ref[...], 0.0)
    mean = jnp.mean(y, axis=-1, keepdims=True)
    var = jnp.mean((y - mean) ** 2, axis=-1, keepdims=True)
    o_ref[...] = (y - mean) / jnp.sqrt(var + 1e-3) * g_ref[...] + b_ref[...]


def _layernorm(ctx, gamma, beta):
    blk = 1024
    return pl.pallas_call(
        _ln_body,
        grid=(N_PAD // blk,),
        in_specs=[
            pl.BlockSpec((blk, HU), lambda i: (i, 0)),
            pl.BlockSpec((1, HU), lambda i: (0, 0)),
            pl.BlockSpec((1, HU), lambda i: (0, 0)),
        ],
        out_specs=pl.BlockSpec((blk, HU), lambda i: (i, 0)),
        out_shape=jax.ShapeDtypeStruct((N_PAD, HU), jnp.float32),
    )(ctx, gamma.reshape(1, HU), beta.reshape(1, HU))


@jax.jit
def kernel(node_features, neighbors, W, a, gamma, beta):
    x = node_features[0]
    x_pad = jnp.pad(x, ((0, N_PAD - N), (0, 0)))
    nbr_pad = jnp.pad(neighbors[0], ((0, N_PAD - N), (0, 0)))

    eye = jnp.eye(H, dtype=jnp.float32)
    a_self_m = (a[:, :U, None] * eye[:, None, :]).reshape(HU, H)
    a_nbr_m = (a[:, U:, None] * eye[:, None, :]).reshape(HU, H)

    h, s_self, s_nbr = _project(x_pad, W, a_self_m, a_nbr_m)
    zeros8 = jnp.zeros((8, HU), jnp.float32)
    hz = jnp.concatenate(
        [zeros8, h[:1], jnp.zeros((7, HU), jnp.float32), h], axis=0)
    s_nbr_g = jnp.concatenate(
        [jnp.full((GUARD,), NEG, jnp.float32), s_nbr.reshape(-1)])
    ctx = _attention(hz, s_self, s_nbr_g, nbr_pad)
    out = _layernorm(ctx, gamma, beta)
    return out[None, :N, :]


# speedup vs baseline: 9.3793x; 9.3793x over previous
import jax
import jax.numpy as jnp
from jax import lax
from jax.experimental import pallas as pl
from jax.experimental.pallas import tpu as pltpu
from jax.experimental.pallas import tpu_sc as plsc

N = 10000
K = 32
D = 128
H = 4
U = 32
HU = H * U

NW = 32
CH = 2
N_PAD = 10240
NODES_W = N_PAD // NW
CHUNKS_W = NODES_W // CH
NBUF = 2
GRP = 4
GUARD = 8
WIN = 5120
NEG = -1000000000.0


def _splat(val, dtype=jnp.float32):
    return jnp.full((16,), val, dtype=dtype)


_GDN = lax.GatherDimensionNumbers(
    offset_dims=(), collapsed_slice_dims=(0,), start_index_map=(0,))


def _gather16(vec, idx16):
    return lax.gather(vec, idx16[:, None], _GDN, (1,),
                      mode=lax.GatherScatterMode.PROMISE_IN_BOUNDS)


def _last_lane(vec):
    return _gather16(vec, _splat(15, jnp.int32))


def _mm_body(x_ref, w_ref, asx_ref, anx_ref, h_ref, ss_ref, sn_ref):
    h = jnp.dot(x_ref[...], w_ref[...], preferred_element_type=jnp.float32)
    h_ref[...] = h
    ss_ref[...] = jnp.dot(h, asx_ref[...], preferred_element_type=jnp.float32)
    sn_ref[...] = jnp.dot(h, anx_ref[...], preferred_element_type=jnp.float32)


def _project(x_pad, W, a_self_m, a_nbr_m):
    blk = 1024
    grid = N_PAD // blk
    return pl.pallas_call(
        _mm_body,
        grid=(grid,),
        in_specs=[
            pl.BlockSpec((blk, D), lambda i: (i, 0)),
            pl.BlockSpec((D, HU), lambda i: (0, 0)),
            pl.BlockSpec((HU, H), lambda i: (0, 0)),
            pl.BlockSpec((HU, H), lambda i: (0, 0)),
        ],
        out_specs=[
            pl.BlockSpec((blk, HU), lambda i: (i, 0)),
            pl.BlockSpec((blk, H), lambda i: (i, 0)),
            pl.BlockSpec((blk, H), lambda i: (i, 0)),
        ],
        out_shape=[
            jax.ShapeDtypeStruct((N_PAD, HU), jnp.float32),
            jax.ShapeDtypeStruct((N_PAD, H), jnp.float32),
            jax.ShapeDtypeStruct((N_PAD, H), jnp.float32),
        ],
    )(x_pad, W, a_self_m, a_nbr_m)


def _attn_body(hz_hbm, ss_hbm, sng_hbm, nbr_hbm, out_hbm,
               sng_v, sself_v, adj_v, isp_refs, ihbm_refs,
               rsp_refs, rhbm_refs, ctx_v, hz_sp, sems_sp, sems_hbm):
    cid = lax.axis_index("c")
    sid = lax.axis_index("s")
    wid = cid * 16 + sid
    node0 = wid * NODES_W
    start_c = cid * WIN

    rows_per_sub = WIN // 16
    pltpu.sync_copy(
        hz_hbm.at[pl.ds(start_c + 72 + sid * rows_per_sub, rows_per_sub)],
        hz_sp.at[pl.ds(64 + sid * rows_per_sub, rows_per_sub)])

    @pl.when(sid < 8)
    def _():
        pltpu.sync_copy(hz_hbm.at[pl.ds(sid * 8, 8)],
                        hz_sp.at[pl.ds(sid * 8, 8)])

    pltpu.sync_copy(sng_hbm, sng_v)
    pltpu.sync_copy(ss_hbm.at[pl.ds(node0 * H, NODES_W * H)], sself_v)
    pltpu.sync_copy(nbr_hbm.at[pl.ds(node0 * K, NODES_W * K)], adj_v)
    plsc.subcore_barrier()

    vstart = _splat(start_c, jnp.int32)
    vend = _splat(start_c + WIN, jnp.int32)
    vzero = _splat(0, jnp.int32)
    v63 = _splat(63, jnp.int32)
    vsp_off = _splat(63 - start_c, jnp.int32)
    vhbm_off = _splat(71, jnp.int32)
    vinv = _splat(64, jnp.int32)

    def issue(g, slot):
        for q in range(CH * 2):
            v = adj_v[pl.ds(g * CH * K + q * 16, 16)]
            in_win = (v > vstart) & (v <= vend)
            vspread = v & v63
            isp = jnp.where(in_win, v + vsp_off, vspread)
            ihbm = jnp.where(in_win, vspread,
                             jnp.where(v > vzero, v + vhbm_off, vinv))
            isp_refs[slot][pl.ds(q * 16, 16)] = isp
            ihbm_refs[slot][pl.ds(q * 16, 16)] = ihbm
        pltpu.make_async_copy(
            hz_sp.at[isp_refs[slot]], rsp_refs[slot], sems_sp[slot]).start()
        pltpu.make_async_copy(
            hz_hbm.at[ihbm_refs[slot]], rhbm_refs[slot], sems_hbm[slot]).start()

    def wait(slot):
        pltpu.make_async_copy(
            hz_sp.at[isp_refs[slot]], rsp_refs[slot], sems_sp[slot]).wait()
        pltpu.make_async_copy(
            hz_hbm.at[ihbm_refs[slot]], rhbm_refs[slot], sems_hbm[slot]).wait()

    def compute(g, slot, jrow0):
        rsp = rsp_refs[slot]
        rhbm = rhbm_refs[slot]
        for j in range(CH):
            nl = g * CH + j
            nbrs = [adj_v[pl.ds(nl * K + kc * 16, 16)] for kc in range(2)]
            evecs = []
            for h in range(H):
                sself = plsc.load_gather(
                    sself_v, [_splat(nl * H + h, jnp.int32)])
                scs = []
                for kc in range(2):
                    snbr = plsc.load_gather(
                        sng_v, [nbrs[kc] * H + (H + h)])
                    sc = sself + snbr
                    scs.append(jnp.where(sc > 0, sc, 0.2 * sc))
                mx = _last_lane(plsc.cummax(jnp.maximum(scs[0], scs[1])))
                e0 = jnp.exp(scs[0] - mx)
                e1 = jnp.exp(scs[1] - mx)
                rden = 1.0 / _last_lane(plsc.cumsum(e0 + e1))
                evecs.append((e0 * rden, e1 * rden))

            def kbody(kc):
                def body(it, accs):
                    base = it * 4
                    lane0 = _splat(base - kc * 16, jnp.int32)
                    out = list(accs)
                    ebs = [[_gather16(evecs[h][kc], lane0 + jj)
                            for jj in range(4)] for h in range(H)]
                    for jj in range(4):
                        row = j * K + base + jj
                        for h in range(H):
                            for uc in range(2):
                                c = h * 2 + uc
                                rv = (rsp[row, pl.ds(c * 16, 16)] +
                                      rhbm[row, pl.ds(c * 16, 16)])
                                out[c] = out[c] + ebs[h][jj] * rv
                    return tuple(out)
                return body

            accs = tuple(jnp.zeros((16,), jnp.float32) for _ in range(8))
            accs = lax.fori_loop(0, 4, kbody(0), accs)
            accs = lax.fori_loop(4, 8, kbody(1), accs)
            for c in range(8):
                ctx_v[jrow0 + j, pl.ds(c * 16, 16)] = accs[c]

    issue(0, 0)

    def outer(t, carry):
        g0 = t * GRP
        for b in range(GRP):
            g = g0 + b
            issue(jnp.minimum(g + 1, CHUNKS_W - 1), (b + 1) % NBUF)
            wait(b % NBUF)
            compute(g, b % NBUF, b * CH)
        pltpu.sync_copy(ctx_v, out_hbm.at[pl.ds(node0 + g0 * CH, GRP * CH)])
        return carry

    lax.fori_loop(0, CHUNKS_W // GRP, outer, 0)
    wait(0)


def _attention(hz, s_self, s_nbr_g, nbr_pad):
    mesh = plsc.VectorSubcoreMesh(core_axis_name="c", subcore_axis_name="s")
    kfn = pl.kernel(
        _attn_body,
        out_type=jax.ShapeDtypeStruct((N_PAD, HU), jnp.float32),
        mesh=mesh,
        scratch_types=[
            pltpu.VMEM((GUARD + N_PAD * H,), jnp.float32),
            pltpu.VMEM((NODES_W * H,), jnp.float32),
            pltpu.VMEM((NODES_W * K,), jnp.int32),
            [pltpu.VMEM((CH * K,), jnp.int32)] * NBUF,
            [pltpu.VMEM((CH * K,), jnp.int32)] * NBUF,
            [pltpu.VMEM((CH * K, HU), jnp.float32)] * NBUF,
            [pltpu.VMEM((CH * K, HU), jnp.float32)] * NBUF,
            pltpu.VMEM((GRP * CH, HU), jnp.float32),
            pltpu.VMEM_SHARED((WIN + 64, HU), jnp.float32),
            [pltpu.SemaphoreType.DMA] * NBUF,
            [pltpu.SemaphoreType.DMA] * NBUF,
        ],
        compiler_params=pltpu.CompilerParams(needs_layout_passes=False),
    )
    return kfn(hz, s_self.reshape(-1), s_nbr_g, nbr_pad.reshape(-1))


def _ln_body(x_ref, g_ref, b_ref, o_ref):
    y = jnp.maximum(x_ref[...], 0.0)
    mean = jnp.mean(y, axis=-1, keepdims=True)
    var = jnp.mean((y - mean) ** 2, axis=-1, keepdims=True)
    o_ref[...] = (y - mean) / jnp.sqrt(var + 1e-3) * g_ref[...] + b_ref[...]


def _layernorm(ctx, gamma, beta):
    blk = 1024
    return pl.pallas_call(
        _ln_body,
        grid=(N_PAD // blk,),
        in_specs=[
            pl.BlockSpec((blk, HU), lambda i: (i, 0)),
            pl.BlockSpec((1, HU), lambda i: (0, 0)),
            pl.BlockSpec((1, HU), lambda i: (0, 0)),
        ],
        out_specs=pl.BlockSpec((blk, HU), lambda i: (i, 0)),
        out_shape=jax.ShapeDtypeStruct((N_PAD, HU), jnp.float32),
    )(ctx, gamma.reshape(1, HU), beta.reshape(1, HU))


@jax.jit
def kernel(node_features, neighbors, W, a, gamma, beta):
    x = node_features[0]
    x_pad = jnp.pad(x, ((0, N_PAD - N), (0, 0)))
    nbr_pad = jnp.pad(neighbors[0], ((0, N_PAD - N), (0, 0)))

    eye = jnp.eye(H, dtype=jnp.float32)
    a_self_m = (a[:, :U, None] * eye[:, None, :]).reshape(HU, H)
    a_nbr_m = (a[:, U:, None] * eye[:, None, :]).reshape(HU, H)

    h, s_self, s_nbr = _project(x_pad, W, a_self_m, a_nbr_m)
    hz = jnp.concatenate(
        [jnp.zeros((64, HU), jnp.float32), h[:1],
         jnp.zeros((7, HU), jnp.float32), h], axis=0)
    s_nbr_g = jnp.concatenate(
        [jnp.full((GUARD,), NEG, jnp.float32), s_nbr.reshape(-1)])
    ctx = _attention(hz, s_self, s_nbr_g, nbr_pad)
    out = _layernorm(ctx, gamma, beta)
    return out[None, :N, :]


# speedup vs baseline: 11.7780x; 1.2557x over previous
import functools

import jax
import jax.numpy as jnp
from jax import lax
from jax.experimental import pallas as pl
from jax.experimental.pallas import tpu as pltpu
from jax.experimental.pallas import tpu_sc as plsc

N = 10000
K = 32
D = 128
H = 4
U = 32
HU = H * U

NW = 32
CH = 4
N_PAD = 10240
NODES_W = N_PAD // NW
CHUNKS_W = NODES_W // CH
NEG = -1000000000.0


def _splat(val, dtype=jnp.float32):
    return jnp.full((16,), val, dtype=dtype)


_GDN = lax.GatherDimensionNumbers(
    offset_dims=(), collapsed_slice_dims=(0,), start_index_map=(0,))


def _gather16(vec, idx16):
    return lax.gather(vec, idx16[:, None], _GDN, (1,),
                      mode=lax.GatherScatterMode.PROMISE_IN_BOUNDS)


def _last_lane(vec):
    return _gather16(vec, _splat(15, jnp.int32))


def _mm_body(x_ref, w_ref, asx_ref, anx_ref, h_ref, ss_ref, sn_ref):
    h = jnp.dot(x_ref[...], w_ref[...], preferred_element_type=jnp.float32)
    h_ref[...] = h
    ss_ref[...] = jnp.dot(h, asx_ref[...], preferred_element_type=jnp.float32)
    sn_ref[...] = jnp.dot(h, anx_ref[...], preferred_element_type=jnp.float32)


def _project(x_pad, W, a_self_m, a_nbr_m):
    blk = 1024
    grid = N_PAD // blk
    return pl.pallas_call(
        _mm_body,
        grid=(grid,),
        in_specs=[
            pl.BlockSpec((blk, D), lambda i: (i, 0)),
            pl.BlockSpec((D, HU), lambda i: (0, 0)),
            pl.BlockSpec((HU, H), lambda i: (0, 0)),
            pl.BlockSpec((HU, H), lambda i: (0, 0)),
        ],
        out_specs=[
            pl.BlockSpec((blk, HU), lambda i: (i, 0)),
            pl.BlockSpec((blk, H), lambda i: (i, 0)),
            pl.BlockSpec((blk, H), lambda i: (i, 0)),
        ],
        out_shape=[
            jax.ShapeDtypeStruct((N_PAD, HU), jnp.float32),
            jax.ShapeDtypeStruct((N_PAD, H), jnp.float32),
            jax.ShapeDtypeStruct((N_PAD, H), jnp.float32),
        ],
    )(x_pad, W, a_self_m, a_nbr_m)


def _attn_body(h_hbm, ss_hbm, sn_hbm, nbr_hbm, gam_hbm, bet_hbm, out_hbm,
               snbr_v, sself_v, adj_v, idx0_v, idx1_v, rows0_v, rows1_v,
               ctx_v, gam_v, bet_v, sem0, sem1):
    cid = lax.axis_index("c")
    sid = lax.axis_index("s")
    wid = cid * 16 + sid
    node0 = wid * NODES_W

    pltpu.sync_copy(sn_hbm, snbr_v)
    pltpu.sync_copy(ss_hbm.at[pl.ds(node0 * H, NODES_W * H)], sself_v)
    pltpu.sync_copy(nbr_hbm.at[pl.ds(node0, NODES_W)], adj_v)
    pltpu.sync_copy(gam_hbm, gam_v)
    pltpu.sync_copy(bet_hbm, bet_v)

    idx_bufs = (idx0_v, idx1_v)
    row_bufs = (rows0_v, rows1_v)
    sems = (sem0, sem1)

    def issue(g, slot):
        for j in range(CH):
            nl = jnp.minimum(g * CH + j, NODES_W - 1)
            for kc in range(2):
                nbr = adj_v[nl, pl.ds(kc * 16, 16)]
                idx = jnp.maximum(nbr - 1, 0)
                idx_bufs[slot][pl.ds(j * K + kc * 16, 16)] = idx
        pltpu.make_async_copy(
            h_hbm.at[idx_bufs[slot]], row_bufs[slot], sems[slot]).start()

    def compute(g, slot):
        rows = row_bufs[slot]
        for j in range(CH):
            nl = g * CH + j
            nbrs = []
            valids = []
            for kc in range(2):
                nbr = adj_v[nl, pl.ds(kc * 16, 16)]
                nbrs.append(jnp.maximum(nbr - 1, 0))
                valids.append(nbr > 0)
            evecs = []
            rinvs = []
            for h in range(H):
                hsplat = _splat(h, jnp.int32)
                scs = []
                sself = plsc.load_gather(
                    sself_v, [_splat(nl * H + h, jnp.int32)])
                for kc in range(2):
                    snbr = plsc.load_gather(snbr_v, [nbrs[kc] * H + h])
                    sc = sself + snbr
                    sc = jnp.where(sc > 0, sc, 0.2 * sc)
                    sc = jnp.where(valids[kc], sc, NEG)
                    scs.append(sc)
                mx = _last_lane(plsc.cummax(jnp.maximum(scs[0], scs[1])))
                e0 = jnp.exp(scs[0] - mx)
                e1 = jnp.exp(scs[1] - mx)
                rden = 1.0 / _last_lane(plsc.cumsum(e0 + e1))
                evecs.append((e0 * rden, e1 * rden))

            def kbody(kc):
                def body(it, accs):
                    base = it * 8
                    lane0 = _splat(base - kc * 16, jnp.int32)
                    out = list(accs)
                    ebs = [[_gather16(evecs[h][kc], lane0 + jj)
                            for jj in range(8)] for h in range(H)]
                    for jj in range(8):
                        row = j * K + base + jj
                        for h in range(H):
                            for uc in range(2):
                                c = h * 2 + uc
                                rv = rows[row, pl.ds(c * 16, 16)]
                                out[c] = out[c] + ebs[h][jj] * rv
                    return tuple(out)
                return body

            accs = tuple(jnp.zeros((16,), jnp.float32) for _ in range(8))
            accs = lax.fori_loop(0, 2, kbody(0), accs)
            accs = lax.fori_loop(2, 4, kbody(1), accs)
            ys = [jnp.maximum(a, 0.0) for a in accs]
            s1 = ys[0]
            s2 = ys[0] * ys[0]
            for c in range(1, 8):
                s1 = s1 + ys[c]
                s2 = s2 + ys[c] * ys[c]
            tot = _last_lane(plsc.cumsum(s1))
            tot2 = _last_lane(plsc.cumsum(s2))
            mean = tot * (1.0 / HU)
            var = tot2 * (1.0 / HU) - mean * mean
            aeps = var + 1e-3
            xi = plsc.bitcast(aeps, jnp.int32)
            mi = _splat(0x5F3759DF, jnp.int32) - (xi >> 1)
            r = plsc.bitcast(mi, jnp.float32)
            half = aeps * 0.5
            for _ in range(3):
                r = r * (1.5 - half * r * r)
            for c in range(8):
                g16 = gam_v[pl.ds(c * 16, 16)]
                b16 = bet_v[pl.ds(c * 16, 16)]
                ctx_v[j, pl.ds(c * 16, 16)] = (ys[c] - mean) * r * g16 + b16
        pltpu.sync_copy(ctx_v, out_hbm.at[pl.ds(node0 + g * CH, CH)])

    issue(0, 0)

    def outer(t, carry):
        g0 = 2 * t
        issue(g0 + 1, 1)
        pltpu.make_async_copy(
            h_hbm.at[idx_bufs[0]], row_bufs[0], sems[0]).wait()
        compute(g0, 0)
        issue(jnp.minimum(g0 + 2, CHUNKS_W - 1), 0)
        pltpu.make_async_copy(
            h_hbm.at[idx_bufs[1]], row_bufs[1], sems[1]).wait()
        compute(g0 + 1, 1)
        return carry

    lax.fori_loop(0, CHUNKS_W // 2, outer, 0)
    pltpu.make_async_copy(h_hbm.at[idx_bufs[0]], row_bufs[0], sems[0]).wait()


def _attention(h, s_self, s_nbr, nbr_pad, gamma, beta):
    mesh = plsc.VectorSubcoreMesh(core_axis_name="c", subcore_axis_name="s")
    kfn = pl.kernel(
        _attn_body,
        out_type=jax.ShapeDtypeStruct((N_PAD, HU), jnp.float32),
        mesh=mesh,
        scratch_types=[
            pltpu.VMEM((N_PAD * H,), jnp.float32),
            pltpu.VMEM((NODES_W * H,), jnp.float32),
            pltpu.VMEM((NODES_W, K), jnp.int32),
            pltpu.VMEM((CH * K,), jnp.int32),
            pltpu.VMEM((CH * K,), jnp.int32),
            pltpu.VMEM((CH * K, HU), jnp.float32),
            pltpu.VMEM((CH * K, HU), jnp.float32),
            pltpu.VMEM((CH, HU), jnp.float32),
            pltpu.VMEM((HU,), jnp.float32),
            pltpu.VMEM((HU,), jnp.float32),
            pltpu.SemaphoreType.DMA,
            pltpu.SemaphoreType.DMA,
        ],
        compiler_params=pltpu.CompilerParams(needs_layout_passes=False),
    )
    return kfn(h, s_self.reshape(-1), s_nbr.reshape(-1), nbr_pad, gamma, beta)


def _ln_body(x_ref, g_ref, b_ref, o_ref):
    y = jnp.maximum(x_ref[...], 0.0)
    mean = jnp.mean(y, axis=-1, keepdims=True)
    var = jnp.mean((y - mean) ** 2, axis=-1, keepdims=True)
    o_ref[...] = (y - mean) / jnp.sqrt(var + 1e-3) * g_ref[...] + b_ref[...]


def _layernorm(ctx, gamma, beta):
    blk = 1024
    return pl.pallas_call(
        _ln_body,
        grid=(N_PAD // blk,),
        in_specs=[
            pl.BlockSpec((blk, HU), lambda i: (i, 0)),
            pl.BlockSpec((1, HU), lambda i: (0, 0)),
            pl.BlockSpec((1, HU), lambda i: (0, 0)),
        ],
        out_specs=pl.BlockSpec((blk, HU), lambda i: (i, 0)),
        out_shape=jax.ShapeDtypeStruct((N_PAD, HU), jnp.float32),
    )(ctx, gamma.reshape(1, HU), beta.reshape(1, HU))


@jax.jit
def kernel(node_features, neighbors, W, a, gamma, beta):
    x = node_features[0]
    x_pad = jnp.pad(x, ((0, N_PAD - N), (0, 0)))
    nbr_pad = jnp.pad(neighbors[0], ((0, N_PAD - N), (0, 0)))

    eye = jnp.eye(H, dtype=jnp.float32)
    a_self_m = (a[:, :U, None] * eye[:, None, :]).reshape(HU, H)
    a_nbr_m = (a[:, U:, None] * eye[:, None, :]).reshape(HU, H)

    h, s_self, s_nbr = _project(x_pad, W, a_self_m, a_nbr_m)
    out = _attention(h, s_self, s_nbr, nbr_pad, gamma, beta)
    return out[None, :N, :]
